# baseline (speedup 1.0000x reference)
import functools

import jax
import jax.numpy as jnp
from jax.experimental import pallas as pl
from jax.experimental.pallas import tpu as pltpu


_INV_SQRT2 = 0.7071067811865476


def _gelu_exact(z):
    return 0.5 * z * (1.0 + jax.lax.erf(z * _INV_SQRT2))


def _dot(a, b):
    return jax.lax.dot_general(a, b, (((1,), (0,)), ((), ())),
                               preferred_element_type=jnp.float32)


def _dotT(a, b):
    return jax.lax.dot_general(a, b, (((1,), (1,)), ((), ())),
                               preferred_element_type=jnp.float32)


def _cast_kernel(wi_ref, wr1_ref, wo_ref, p_ref,
                 wi_o, wr1_o, wo_o, p_o):
    wi_o[...] = wi_ref[...].astype(jnp.bfloat16)
    wr1_o[...] = wr1_ref[...].astype(jnp.bfloat16)
    wo_o[...] = wo_ref[...].astype(jnp.bfloat16)
    p_o[...] = p_ref[...].astype(jnp.bfloat16)


def _fused_kernel(x_ref, wi_ref, bi_ref, wr1_ref, br1_ref, wr2_ref, br2_ref,
                  wo_ref, bo_ref, gamma_ref, beta_ref,
                  p_ref, conf_ref, evid_ref, age_ref,
                  o_ref, colscale_ref, addrow_ref, *, protos_per_type):
    D = x_ref.shape[1]

    @pl.when(pl.program_id(1) == 0)
    def _():
        pb = p_ref[...]
        pp = pb * pb
        ones = jnp.ones((1, D), dtype=jnp.bfloat16)
        sq = _dotT(ones, pp)
        inv_p = jax.lax.rsqrt(jnp.maximum(sq, 1e-24))
        conf = conf_ref[...]
        evid = evid_ref[...]
        age = age_ref[...]
        colscale_ref[...] = inv_p * conf * (0.45 / 0.07)
        recency = jnp.exp(age * (-1.0 / 200.0))
        freq = jnp.log(evid + 1.0) / (jnp.log(jnp.max(evid) + 2.0) + 1e-08)
        addrow_ref[...] = (0.2 * recency + 0.15 * freq + 0.1 * conf
                           + 0.1 * 0.9) * (1.0 / 0.07)

    x = x_ref[...]
    xb = x.astype(jnp.bfloat16)

    h = _dotT(xb, wi_ref[...]) + bi_ref[...]
    inv_hn = jax.lax.rsqrt(jnp.maximum(jnp.sum(h * h, axis=1, keepdims=True),
                                       1e-24))
    hb = (h * inv_hn).astype(jnp.bfloat16)

    r = _gelu_exact(_dotT(xb, wr1_ref[...]) + br1_ref[...])
    tl = _dotT(r.astype(jnp.bfloat16), wr2_ref[...]) + br2_ref[...]
    te = jnp.exp(tl - jnp.max(tl, axis=1, keepdims=True))
    tw = te * (1.0 / jnp.sum(te, axis=1, keepdims=True))

    s0 = _dotT(hb, p_ref[...])
    z_x = _dotT(xb, wo_ref[:, :D])

    sim = s0 * colscale_ref[...]
    n_types = tl.shape[1]
    tmask = jnp.concatenate(
        [jnp.broadcast_to(tw[:, t:t + 1], (tw.shape[0], protos_per_type))
         for t in range(n_types)], axis=1)
    e = jnp.exp(jnp.clip(sim * tmask + addrow_ref[...], 0.0, 1.0 / 0.07))
    attn = e * (1.0 / jnp.sum(e, axis=1, keepdims=True))

    retr = _dot(attn.astype(jnp.bfloat16), p_ref[...])
    z = z_x + _dotT(retr.astype(jnp.bfloat16), wo_ref[:, D:])
    out = _gelu_exact(z + bo_ref[...])

    y = out + x
    mu = jnp.mean(y, axis=1, keepdims=True)
    yc = y - mu
    var = jnp.mean(yc * yc, axis=1, keepdims=True)
    o_ref[...] = yc * jax.lax.rsqrt(var + 1e-05) * gamma_ref[...] + beta_ref[...]


def kernel(x, Wi, bi, Wr1, br1, Wr2, br2, Wo, bo, gamma, beta,
           prototypes, proto_conf, proto_evidence, proto_age):
    B, D = x.shape
    P = prototypes.shape[0]
    n_types = Wr2.shape[0]
    protos_per_type = P // n_types
    BM = 512
    assert B % BM == 0

    bf = jnp.bfloat16
    CG = 16
    wi_b, wr1_b, wo_b, p_b = pl.pallas_call(
        _cast_kernel,
        grid=(CG,),
        in_specs=[
            pl.BlockSpec((D // CG, D), lambda i: (i, 0)),
            pl.BlockSpec((D // 2 // CG, D), lambda i: (i, 0)),
            pl.BlockSpec((D // CG, 2 * D), lambda i: (i, 0)),
            pl.BlockSpec((P // CG, D), lambda i: (i, 0)),
        ],
        out_specs=[
            pl.BlockSpec((D // CG, D), lambda i: (i, 0)),
            pl.BlockSpec((D // 2 // CG, D), lambda i: (i, 0)),
            pl.BlockSpec((D // CG, 2 * D), lambda i: (i, 0)),
            pl.BlockSpec((P // CG, D), lambda i: (i, 0)),
        ],
        out_shape=[
            jax.ShapeDtypeStruct(Wi.shape, bf),
            jax.ShapeDtypeStruct(Wr1.shape, bf),
            jax.ShapeDtypeStruct(Wo.shape, bf),
            jax.ShapeDtypeStruct(prototypes.shape, bf),
        ],
        compiler_params=pltpu.CompilerParams(
            dimension_semantics=("parallel",),
        ),
    )(Wi, Wr1, Wo, prototypes)
    wr2_b = Wr2.astype(bf)

    row = lambda v: v.reshape(1, -1).astype(jnp.float32)
    bi_r, br1_r, br2_r, bo_r = row(bi), row(br1), row(br2), row(bo)
    gamma_r, beta_r = row(gamma), row(beta)
    conf_r, evid_r, age_r = row(proto_conf), row(proto_evidence), row(proto_age)

    full = lambda a: pl.BlockSpec(a.shape, lambda c, j: (0,) * a.ndim)
    NC = 2
    NJ = B // BM // NC
    grid = (NC, NJ)
    xmap = lambda c, j: (c * NJ + j, 0)

    return pl.pallas_call(
        functools.partial(_fused_kernel, protos_per_type=protos_per_type),
        grid=grid,
        in_specs=[
            pl.BlockSpec((BM, D), xmap),
            full(wi_b), full(bi_r), full(wr1_b), full(br1_r),
            full(wr2_b), full(br2_r), full(wo_b), full(bo_r),
            full(gamma_r), full(beta_r), full(p_b),
            full(conf_r), full(evid_r), full(age_r),
        ],
        out_specs=pl.BlockSpec((BM, D), xmap),
        out_shape=jax.ShapeDtypeStruct((B, D), jnp.float32),
        scratch_shapes=[
            pltpu.VMEM((1, P), jnp.float32),
            pltpu.VMEM((1, P), jnp.float32),
        ],
        compiler_params=pltpu.CompilerParams(
            dimension_semantics=("parallel", "arbitrary"),
            vmem_limit_bytes=100 * 1024 * 1024,
        ),
    )(x, wi_b, bi_r, wr1_b, br1_r, wr2_b, br2_r, wo_b, bo_r,
      gamma_r, beta_r, p_b, conf_r, evid_r, age_r)


# speedup vs baseline: 1.0195x; 1.0195x over previous
import functools

import jax
import jax.numpy as jnp
from jax.experimental import pallas as pl
from jax.experimental.pallas import tpu as pltpu


_INV_SQRT2 = 0.7071067811865476
_CH = 128


def _gelu_exact(z):
    return 0.5 * z * (1.0 + jax.lax.erf(z * _INV_SQRT2))


def _dot(a, b):
    return jax.lax.dot_general(a, b, (((1,), (0,)), ((), ())),
                               preferred_element_type=jnp.float32)


def _dotT(a, b):
    return jax.lax.dot_general(a, b, (((1,), (1,)), ((), ())),
                               preferred_element_type=jnp.float32)


def _fused_kernel(x_ref, wi_hbm, bi_ref, wr1_hbm, br1_ref, wr2_ref, br2_ref,
                  wo_hbm, bo_ref, gamma_ref, beta_ref,
                  p_hbm, conf_ref, evid_ref, age_ref,
                  o_ref, wi_s, wr1_s, wo_s, p_s,
                  colscale_ref, addrow_ref, stage, sems, *, protos_per_type):
    D = x_ref.shape[1]

    @pl.when(pl.program_id(0) == 0)
    def _():
        chunks = []
        for hbm, dst in ((wi_hbm, wi_s), (wr1_hbm, wr1_s),
                         (wo_hbm, wo_s), (p_hbm, p_s)):
            rows, lanes = hbm.shape
            for off in range(0, rows, _CH):
                chunks.append((hbm, dst, off, lanes))

        def _start(k, slot):
            hbm, _, off, lanes = chunks[k]
            pltpu.make_async_copy(
                hbm.at[pl.ds(off, _CH), :],
                stage.at[slot, :, pl.ds(0, lanes)],
                sems.at[slot],
            ).start()

        n = len(chunks)
        _start(0, 0)
        if n > 1:
            _start(1, 1)
        for k in range(n):
            slot = k % 2
            hbm, dst, off, lanes = chunks[k]
            pltpu.make_async_copy(
                hbm.at[pl.ds(off, _CH), :],
                stage.at[slot, :, pl.ds(0, lanes)],
                sems.at[slot],
            ).wait()
            dst[pl.ds(off, _CH), :] = (
                stage[slot, :, pl.ds(0, lanes)].astype(jnp.bfloat16))
            if k + 2 < n:
                _start(k + 2, slot)

        pb = p_s[...]
        pp = pb * pb
        ones = jnp.ones((1, D), dtype=jnp.bfloat16)
        sq = _dotT(ones, pp)
        inv_p = jax.lax.rsqrt(jnp.maximum(sq, 1e-24))
        conf = conf_ref[...]
        colscale_ref[...] = inv_p * conf * (0.45 / 0.07)
        recency = jnp.exp(age_ref[...] * (-1.0 / 200.0))
        evid = evid_ref[...]
        freq = jnp.log(evid + 1.0) / (jnp.log(jnp.max(evid) + 2.0) + 1e-08)
        addrow_ref[...] = (0.2 * recency + 0.15 * freq + 0.1 * conf
                           + 0.1 * 0.9) * (1.0 / 0.07)

    x = x_ref[...]
    xb = x.astype(jnp.bfloat16)

    h = _dotT(xb, wi_s[...]) + bi_ref[...]
    inv_hn = jax.lax.rsqrt(jnp.maximum(jnp.sum(h * h, axis=1, keepdims=True),
                                       1e-24))
    hb = (h * inv_hn).astype(jnp.bfloat16)

    r = _gelu_exact(_dotT(xb, wr1_s[...]) + br1_ref[...])
    tl = _dotT(r.astype(jnp.bfloat16), wr2_ref[...]) + br2_ref[...]
    te = jnp.exp(tl - jnp.max(tl, axis=1, keepdims=True))
    tw = te * (1.0 / jnp.sum(te, axis=1, keepdims=True))

    s0 = _dotT(hb, p_s[...])
    z_x = _dotT(xb, wo_s[:, :D])

    sim = s0 * colscale_ref[...]
    n_types = tl.shape[1]
    tmask = jnp.concatenate(
        [jnp.broadcast_to(tw[:, t:t + 1], (tw.shape[0], protos_per_type))
         for t in range(n_types)], axis=1)
    e = jnp.exp(jnp.clip(sim * tmask + addrow_ref[...], 0.0, 1.0 / 0.07))
    attn = e * (1.0 / jnp.sum(e, axis=1, keepdims=True))

    retr = _dot(attn.astype(jnp.bfloat16), p_s[...])
    z = z_x + _dotT(retr.astype(jnp.bfloat16), wo_s[:, D:])
    out = _gelu_exact(z + bo_ref[...])

    y = out + x
    mu = jnp.mean(y, axis=1, keepdims=True)
    yc = y - mu
    var = jnp.mean(yc * yc, axis=1, keepdims=True)
    o_ref[...] = yc * jax.lax.rsqrt(var + 1e-05) * gamma_ref[...] + beta_ref[...]


def kernel(x, Wi, bi, Wr1, br1, Wr2, br2, Wo, bo, gamma, beta,
           prototypes, proto_conf, proto_evidence, proto_age):
    B, D = x.shape
    P = prototypes.shape[0]
    n_types = Wr2.shape[0]
    protos_per_type = P // n_types
    BM = 512
    assert B % BM == 0
    for rows in (D, D // 2, P):
        assert rows % _CH == 0

    bf = jnp.bfloat16
    wr2_b = Wr2.astype(bf)

    row = lambda v: v.reshape(1, -1).astype(jnp.float32)
    bi_r, br1_r, br2_r, bo_r = row(bi), row(br1), row(br2), row(bo)
    gamma_r, beta_r = row(gamma), row(beta)
    conf_r, evid_r, age_r = row(proto_conf), row(proto_evidence), row(proto_age)

    full = lambda a: pl.BlockSpec(a.shape, lambda i: (0,) * a.ndim)
    hbm = pl.BlockSpec(memory_space=pl.ANY)
    grid = (B // BM,)

    return pl.pallas_call(
        functools.partial(_fused_kernel, protos_per_type=protos_per_type),
        grid=grid,
        in_specs=[
            pl.BlockSpec((BM, D), lambda i: (i, 0)),
            hbm, full(bi_r), hbm, full(br1_r),
            full(wr2_b), full(br2_r), hbm, full(bo_r),
            full(gamma_r), full(beta_r), hbm,
            full(conf_r), full(evid_r), full(age_r),
        ],
        out_specs=pl.BlockSpec((BM, D), lambda i: (i, 0)),
        out_shape=jax.ShapeDtypeStruct((B, D), jnp.float32),
        scratch_shapes=[
            pltpu.VMEM(Wi.shape, bf),
            pltpu.VMEM(Wr1.shape, bf),
            pltpu.VMEM(Wo.shape, bf),
            pltpu.VMEM(prototypes.shape, bf),
            pltpu.VMEM((1, P), jnp.float32),
            pltpu.VMEM((1, P), jnp.float32),
            pltpu.VMEM((2, _CH, 2 * D), jnp.float32),
            pltpu.SemaphoreType.DMA((2,)),
        ],
        compiler_params=pltpu.CompilerParams(
            dimension_semantics=("arbitrary",),
            vmem_limit_bytes=100 * 1024 * 1024,
        ),
    )(x, Wi, bi_r, Wr1, br1_r, wr2_b, br2_r, Wo, bo_r,
      gamma_r, beta_r, prototypes, conf_r, evid_r, age_r)


# speedup vs baseline: 1.0208x; 1.0012x over previous
import functools

import jax
import jax.numpy as jnp
from jax.experimental import pallas as pl
from jax.experimental.pallas import tpu as pltpu


_INV_SQRT2 = 0.7071067811865476
_CH = 128


def _gelu_exact(z):
    return 0.5 * z * (1.0 + jax.lax.erf(z * _INV_SQRT2))


def _dot(a, b):
    return jax.lax.dot_general(a, b, (((1,), (0,)), ((), ())),
                               preferred_element_type=jnp.float32)


def _dotT(a, b):
    return jax.lax.dot_general(a, b, (((1,), (1,)), ((), ())),
                               preferred_element_type=jnp.float32)


def _fused_kernel(x_ref, wi_hbm, bi_ref, wr1_hbm, br1_ref, wr2_ref, br2_ref,
                  wo_hbm, bo_ref, gamma_ref, beta_ref,
                  p_hbm, conf_ref, evid_ref, age_ref,
                  o_ref, wi_s, wr1_s, wo_s, p_s,
                  colscale_ref, addrow_ref, stage, sems, *, protos_per_type):
    D = x_ref.shape[1]

    @pl.when(pl.program_id(0) == 0)
    def _():
        chunks = []
        for hbm, dst in ((wi_hbm, wi_s), (wr1_hbm, wr1_s),
                         (wo_hbm, wo_s), (p_hbm, p_s)):
            rows, lanes = hbm.shape
            for off in range(0, rows, _CH):
                chunks.append((hbm, dst, off, lanes))

        def _start(k, slot):
            hbm, _, off, lanes = chunks[k]
            pltpu.make_async_copy(
                hbm.at[pl.ds(off, _CH), :],
                stage.at[slot, :, pl.ds(0, lanes)],
                sems.at[slot],
            ).start()

        n = len(chunks)
        _start(0, 0)
        if n > 1:
            _start(1, 1)
        for k in range(n):
            slot = k % 2
            hbm, dst, off, lanes = chunks[k]
            pltpu.make_async_copy(
                hbm.at[pl.ds(off, _CH), :],
                stage.at[slot, :, pl.ds(0, lanes)],
                sems.at[slot],
            ).wait()
            dst[pl.ds(off, _CH), :] = (
                stage[slot, :, pl.ds(0, lanes)].astype(jnp.bfloat16))
            if k + 2 < n:
                _start(k + 2, slot)

        pb = p_s[...]
        pp = pb * pb
        ones = jnp.ones((1, D), dtype=jnp.bfloat16)
        sq = _dotT(ones, pp)
        inv_p = jax.lax.rsqrt(jnp.maximum(sq, 1e-24))
        conf = conf_ref[...]
        colscale_ref[...] = inv_p * conf * (0.45 / 0.07)
        recency = jnp.exp(age_ref[...] * (-1.0 / 200.0))
        evid = evid_ref[...]
        freq = jnp.log(evid + 1.0) / (jnp.log(jnp.max(evid) + 2.0) + 1e-08)
        addrow_ref[...] = (0.2 * recency + 0.15 * freq + 0.1 * conf
                           + 0.1 * 0.9) * (1.0 / 0.07)

    x = x_ref[...]
    xb = x.astype(jnp.bfloat16)

    h = _dotT(xb, wi_s[...]) + bi_ref[...]
    inv_hn = jax.lax.rsqrt(jnp.maximum(jnp.sum(h * h, axis=1, keepdims=True),
                                       1e-24))
    hb = h.astype(jnp.bfloat16)

    r = _gelu_exact(_dotT(xb, wr1_s[...]) + br1_ref[...])
    tl = _dotT(r.astype(jnp.bfloat16), wr2_ref[...]) + br2_ref[...]
    te = jnp.exp(tl - jnp.max(tl, axis=1, keepdims=True))
    tw = te * (1.0 / jnp.sum(te, axis=1, keepdims=True))

    s0 = _dotT(hb, p_s[...])
    z_x = _dotT(xb, wo_s[:, :D])

    sim = s0 * colscale_ref[...] * inv_hn
    n_types = tl.shape[1]
    tmask = jnp.concatenate(
        [jnp.broadcast_to(tw[:, t:t + 1], (tw.shape[0], protos_per_type))
         for t in range(n_types)], axis=1)
    e = jnp.exp(jnp.clip(sim * tmask + addrow_ref[...], 0.0, 1.0 / 0.07))
    attn = e * (1.0 / jnp.sum(e, axis=1, keepdims=True))

    retr = _dot(attn.astype(jnp.bfloat16), p_s[...])
    z = z_x + _dotT(retr.astype(jnp.bfloat16), wo_s[:, D:])
    out = _gelu_exact(z + bo_ref[...])

    y = out + x
    mu = jnp.mean(y, axis=1, keepdims=True)
    yc = y - mu
    var = jnp.mean(yc * yc, axis=1, keepdims=True)
    o_ref[...] = yc * jax.lax.rsqrt(var + 1e-05) * gamma_ref[...] + beta_ref[...]


def kernel(x, Wi, bi, Wr1, br1, Wr2, br2, Wo, bo, gamma, beta,
           prototypes, proto_conf, proto_evidence, proto_age):
    B, D = x.shape
    P = prototypes.shape[0]
    n_types = Wr2.shape[0]
    protos_per_type = P // n_types
    BM = 512
    assert B % BM == 0
    for rows in (D, D // 2, P):
        assert rows % _CH == 0

    bf = jnp.bfloat16
    wr2_b = Wr2.astype(bf)

    row = lambda v: v.reshape(1, -1).astype(jnp.float32)
    bi_r, br1_r, br2_r, bo_r = row(bi), row(br1), row(br2), row(bo)
    gamma_r, beta_r = row(gamma), row(beta)
    conf_r, evid_r, age_r = row(proto_conf), row(proto_evidence), row(proto_age)

    full = lambda a: pl.BlockSpec(a.shape, lambda i: (0,) * a.ndim)
    hbm = pl.BlockSpec(memory_space=pl.ANY)
    grid = (B // BM,)

    return pl.pallas_call(
        functools.partial(_fused_kernel, protos_per_type=protos_per_type),
        grid=grid,
        in_specs=[
            pl.BlockSpec((BM, D), lambda i: (i, 0)),
            hbm, full(bi_r), hbm, full(br1_r),
            full(wr2_b), full(br2_r), hbm, full(bo_r),
            full(gamma_r), full(beta_r), hbm,
            full(conf_r), full(evid_r), full(age_r),
        ],
        out_specs=pl.BlockSpec((BM, D), lambda i: (i, 0)),
        out_shape=jax.ShapeDtypeStruct((B, D), jnp.float32),
        scratch_shapes=[
            pltpu.VMEM(Wi.shape, bf),
            pltpu.VMEM(Wr1.shape, bf),
            pltpu.VMEM(Wo.shape, bf),
            pltpu.VMEM(prototypes.shape, bf),
            pltpu.VMEM((1, P), jnp.float32),
            pltpu.VMEM((1, P), jnp.float32),
            pltpu.VMEM((2, _CH, 2 * D), jnp.float32),
            pltpu.SemaphoreType.DMA((2,)),
        ],
        compiler_params=pltpu.CompilerParams(
            dimension_semantics=("arbitrary",),
            vmem_limit_bytes=100 * 1024 * 1024,
        ),
    )(x, Wi, bi_r, Wr1, br1_r, wr2_b, br2_r, Wo, bo_r,
      gamma_r, beta_r, prototypes, conf_r, evid_r, age_r)


# speedup vs baseline: 1.0354x; 1.0143x over previous
import functools

import jax
import jax.numpy as jnp
from jax.experimental import pallas as pl
from jax.experimental.pallas import tpu as pltpu


_INV_SQRT2 = 0.7071067811865476
_CH = 128


def _gelu_exact(z):
    return 0.5 * z * (1.0 + jax.lax.erf(z * _INV_SQRT2))


def _dot(a, b):
    return jax.lax.dot_general(a, b, (((1,), (0,)), ((), ())),
                               preferred_element_type=jnp.float32)


def _dotT(a, b):
    return jax.lax.dot_general(a, b, (((1,), (1,)), ((), ())),
                               preferred_element_type=jnp.float32)


def _fused_kernel(x_ref, wi_hbm, bi_ref, wr1_hbm, br1_ref, wr2_ref, br2_ref,
                  wo_hbm, bo_ref, gamma_ref, beta_ref,
                  p_hbm, conf_ref, evid_ref, age_ref,
                  o_ref, wi_s, wr1_s, wo_s, p_s,
                  colscale_ref, addrow_ref, stage, sems, *, protos_per_type):
    D = x_ref.shape[1]

    @pl.when(pl.program_id(0) == 0)
    def _():
        chunks = []
        for hbm, dst in ((wi_hbm, wi_s), (wr1_hbm, wr1_s),
                         (wo_hbm, wo_s), (p_hbm, p_s)):
            rows, lanes = hbm.shape
            for off in range(0, rows, _CH):
                chunks.append((hbm, dst, off, lanes))

        def _start(k, slot):
            hbm, _, off, lanes = chunks[k]
            pltpu.make_async_copy(
                hbm.at[pl.ds(off, _CH), :],
                stage.at[slot, :, pl.ds(0, lanes)],
                sems.at[slot],
            ).start()

        n = len(chunks)
        _start(0, 0)
        if n > 1:
            _start(1, 1)
        for k in range(n):
            slot = k % 2
            hbm, dst, off, lanes = chunks[k]
            pltpu.make_async_copy(
                hbm.at[pl.ds(off, _CH), :],
                stage.at[slot, :, pl.ds(0, lanes)],
                sems.at[slot],
            ).wait()
            dst[pl.ds(off, _CH), :] = (
                stage[slot, :, pl.ds(0, lanes)].astype(jnp.bfloat16))
            if k + 2 < n:
                _start(k + 2, slot)

        pb = p_s[...]
        pp = pb * pb
        ones = jnp.ones((1, D), dtype=jnp.bfloat16)
        sq = _dotT(ones, pp)
        inv_p = jax.lax.rsqrt(jnp.maximum(sq, 1e-24))
        conf = conf_ref[...]
        colscale_ref[...] = inv_p * conf * (0.45 / 0.07)
        recency = jnp.exp(age_ref[...] * (-1.0 / 200.0))
        evid = evid_ref[...]
        freq = jnp.log(evid + 1.0) / (jnp.log(jnp.max(evid) + 2.0) + 1e-08)
        addrow_ref[...] = (0.2 * recency + 0.15 * freq + 0.1 * conf
                           + 0.1 * 0.9) * (1.0 / 0.07)

    x = x_ref[...]
    xb = x.astype(jnp.bfloat16)

    h = _dotT(xb, wi_s[...]) + bi_ref[...]
    inv_hn = jax.lax.rsqrt(jnp.maximum(jnp.sum(h * h, axis=1, keepdims=True),
                                       1e-24))
    hb = h.astype(jnp.bfloat16)

    r = _gelu_exact(_dotT(xb, wr1_s[...]) + br1_ref[...])
    tl = _dotT(r.astype(jnp.bfloat16), wr2_ref[...]) + br2_ref[...]
    te = jnp.exp(tl - jnp.max(tl, axis=1, keepdims=True))
    tw = te * (1.0 / jnp.sum(te, axis=1, keepdims=True))

    s0 = _dotT(hb, p_s[...])
    z_x = _dotT(xb, wo_s[:, :D])

    sim = s0 * colscale_ref[...] * inv_hn
    n_types = tl.shape[1]
    tmask = jnp.concatenate(
        [jnp.broadcast_to(tw[:, t:t + 1], (tw.shape[0], protos_per_type))
         for t in range(n_types)], axis=1)
    e = jnp.exp(jnp.clip(sim * tmask + addrow_ref[...], 0.0, 1.0 / 0.07))
    inv_se = 1.0 / jnp.sum(e, axis=1, keepdims=True)

    retr_u = _dot(e.astype(jnp.bfloat16), p_s[...])
    z = z_x + _dotT(retr_u.astype(jnp.bfloat16), wo_s[:, D:]) * inv_se
    out = _gelu_exact(z + bo_ref[...])

    y = out + x
    mu = jnp.mean(y, axis=1, keepdims=True)
    yc = y - mu
    var = jnp.mean(yc * yc, axis=1, keepdims=True)
    o_ref[...] = yc * jax.lax.rsqrt(var + 1e-05) * gamma_ref[...] + beta_ref[...]


def kernel(x, Wi, bi, Wr1, br1, Wr2, br2, Wo, bo, gamma, beta,
           prototypes, proto_conf, proto_evidence, proto_age):
    B, D = x.shape
    P = prototypes.shape[0]
    n_types = Wr2.shape[0]
    protos_per_type = P // n_types
    BM = 512
    assert B % BM == 0
    for rows in (D, D // 2, P):
        assert rows % _CH == 0

    bf = jnp.bfloat16
    wr2_b = Wr2.astype(bf)

    row = lambda v: v.reshape(1, -1).astype(jnp.float32)
    bi_r, br1_r, br2_r, bo_r = row(bi), row(br1), row(br2), row(bo)
    gamma_r, beta_r = row(gamma), row(beta)
    conf_r, evid_r, age_r = row(proto_conf), row(proto_evidence), row(proto_age)

    full = lambda a: pl.BlockSpec(a.shape, lambda i: (0,) * a.ndim)
    hbm = pl.BlockSpec(memory_space=pl.ANY)
    grid = (B // BM,)

    return pl.pallas_call(
        functools.partial(_fused_kernel, protos_per_type=protos_per_type),
        grid=grid,
        in_specs=[
            pl.BlockSpec((BM, D), lambda i: (i, 0)),
            hbm, full(bi_r), hbm, full(br1_r),
            full(wr2_b), full(br2_r), hbm, full(bo_r),
            full(gamma_r), full(beta_r), hbm,
            full(conf_r), full(evid_r), full(age_r),
        ],
        out_specs=pl.BlockSpec((BM, D), lambda i: (i, 0)),
        out_shape=jax.ShapeDtypeStruct((B, D), jnp.float32),
        scratch_shapes=[
            pltpu.VMEM(Wi.shape, bf),
            pltpu.VMEM(Wr1.shape, bf),
            pltpu.VMEM(Wo.shape, bf),
            pltpu.VMEM(prototypes.shape, bf),
            pltpu.VMEM((1, P), jnp.float32),
            pltpu.VMEM((1, P), jnp.float32),
            pltpu.VMEM((2, _CH, 2 * D), jnp.float32),
            pltpu.SemaphoreType.DMA((2,)),
        ],
        compiler_params=pltpu.CompilerParams(
            dimension_semantics=("arbitrary",),
            vmem_limit_bytes=100 * 1024 * 1024,
        ),
    )(x, Wi, bi_r, Wr1, br1_r, wr2_b, br2_r, Wo, bo_r,
      gamma_r, beta_r, prototypes, conf_r, evid_r, age_r)
